# baseline (speedup 1.0000x reference)
import functools

import jax
import jax.numpy as jnp
from jax import lax
from jax.experimental import pallas as pl
from jax.experimental.pallas import tpu as pltpu
from jax.experimental.pallas import tpu_sc as plsc

NUM_FIELDS = 26
VOCAB = 100000
EMB_DIM = 16
DENSE_DIM = 13
BATCH = 4096
H1, H2 = 64, 32
BN_EPS = 1e-5

PLANES = NUM_FIELDS * EMB_DIM
_NC, _NS = 2, 16
_CHUNK = 2048
_NCHUNK = 49
_TAIL_OFF = 48 * _CHUNK


_SLOTS = 160


def _sc_gather(tab3, lin_tab, vsp, bsp):
    mesh = plsc.VectorSubcoreMesh(core_axis_name="c", subcore_axis_name="s")
    fbins = _NCHUNK * _SLOTS

    @functools.partial(
        pl.kernel,
        mesh=mesh,
        compiler_params=pltpu.CompilerParams(needs_layout_passes=False),
        out_type=(
            jax.ShapeDtypeStruct((PLANES, BATCH), jnp.float32),
            jax.ShapeDtypeStruct((NUM_FIELDS, BATCH), jnp.float32),
        ),
        scratch_types=[
            pltpu.VMEM((EMB_DIM, _CHUNK), jnp.float32),
            pltpu.VMEM((_CHUNK,), jnp.float32),
            pltpu.VMEM((fbins,), jnp.int32),
            pltpu.VMEM((fbins,), jnp.int32),
            pltpu.VMEM((EMB_DIM, BATCH), jnp.float32),
            pltpu.VMEM((BATCH,), jnp.float32),
            pltpu.SemaphoreType.DMA,
        ],
    )
    def k(tab, ltab, vs_h, bs_h, x_out, lin_out,
          buf, lbuf, vs_v, bs_v, slab, lslab, sem):
        wid = lax.axis_index("s") * _NC + lax.axis_index("c")

        @pl.when(wid < NUM_FIELDS)
        def _():
            f = wid
            pltpu.sync_copy(vs_h.at[pl.ds(f * fbins, fbins)], vs_v)
            pltpu.sync_copy(bs_h.at[pl.ds(f * fbins, fbins)], bs_v)

            def chunk_body(c, _):
                off = jnp.where(c < _NCHUNK - 1, c * _CHUNK, _TAIL_OFF)
                cp = pltpu.async_copy(tab.at[f, :, pl.ds(off, _CHUNK)],
                                      buf, sem)
                cp2 = pltpu.async_copy(ltab.at[f, pl.ds(off, _CHUNK)],
                                       lbuf, sem)
                cp.wait()
                cp2.wait()
                base = c * _SLOTS
                for g in range(_SLOTS // 16):
                    v_vec = vs_v[pl.ds(base + g * 16, 16)]
                    b_vec = bs_v[pl.ds(base + g * 16, 16)]
                    msk = v_vec >= 0
                    vloc = v_vec - off
                    lvals = plsc.load_gather(lbuf, [vloc], mask=msk)
                    plsc.store_scatter(lslab, [b_vec], lvals, mask=msk)
                    for d in range(EMB_DIM):
                        dv = jnp.full((16,), d, jnp.int32)
                        vals = plsc.load_gather(buf, [dv, vloc], mask=msk)
                        plsc.store_scatter(slab, [dv, b_vec], vals,
                                           mask=msk)
                return _

            lax.fori_loop(0, _NCHUNK, chunk_body, None)
            pltpu.sync_copy(slab, x_out.at[pl.ds(f * EMB_DIM, EMB_DIM)])
            pltpu.sync_copy(lslab, lin_out.at[f])

    return k(tab3, lin_tab, vsp, bsp)


def _tc_body(x_ref, d_ref, lin_ref, sel_ref, wd_ref, w1_ref, b1_ref,
             g1_ref, bt1_ref, w2_ref, b2_ref, g2_ref, bt2_ref, wout_ref,
             cbias_ref, out_ref):
    f32 = jnp.float32
    x = x_ref[...]
    d = d_ref[...]
    sel = sel_ref[...]

    sv = jnp.dot(sel, x, preferred_element_type=f32)
    sq = jnp.dot(sel, x * x, preferred_element_type=f32)
    fm_logit = 0.5 * jnp.sum(sv * sv - sq, axis=0, keepdims=True)

    lin_logit = jnp.sum(lin_ref[...], axis=0, keepdims=True)
    lin_logit = lin_logit + jnp.dot(wd_ref[...], d,
                                    preferred_element_type=f32)

    inv = lax.rsqrt(jnp.float32(1.0 + BN_EPS))
    w1 = w1_ref[...]
    z = jnp.dot(w1[:, :PLANES], x, preferred_element_type=f32)
    z = z + jnp.dot(w1[:, PLANES:], d, preferred_element_type=f32)
    h = jnp.maximum((z + b1_ref[...]) * (g1_ref[...] * inv) + bt1_ref[...],
                    0.0)
    z2 = jnp.dot(w2_ref[...], h, preferred_element_type=f32)
    h2 = jnp.maximum((z2 + b2_ref[...]) * (g2_ref[...] * inv) + bt2_ref[...],
                     0.0)
    dnn_logit = jnp.dot(wout_ref[...], h2, preferred_element_type=f32)

    total = lin_logit + fm_logit + dnn_logit + cbias_ref[...]
    out_ref[...] = jax.nn.sigmoid(total)


def _tc_dense(x_t, d_t, lin_t, sel, wd, w1, b1, g1, bt1, w2, b2, g2, bt2,
              wout, cbias):
    bm = 1024
    grid = (BATCH // bm,)
    full = lambda shape: pl.BlockSpec(shape, lambda i: (0,) * len(shape))
    col = lambda rows: pl.BlockSpec((rows, bm), lambda i: (0, i))
    return pl.pallas_call(
        _tc_body,
        grid=grid,
        in_specs=[
            col(PLANES),
            col(DENSE_DIM),
            col(NUM_FIELDS),
            full((EMB_DIM, PLANES)),
            full((1, DENSE_DIM)),
            full((H1, PLANES + DENSE_DIM)),
            full((H1, 1)), full((H1, 1)), full((H1, 1)),
            full((H2, H1)),
            full((H2, 1)), full((H2, 1)), full((H2, 1)),
            full((1, H2)),
            full((1, 1)),
        ],
        out_specs=col(1),
        out_shape=jax.ShapeDtypeStruct((1, BATCH), jnp.float32),
    )(x_t, d_t, lin_t, sel, wd, w1, b1, g1, bt1, w2, b2, g2, bt2, wout,
      cbias)


def _inverse_perm(dest, fbins):
    def row(d):
        return jnp.full((fbins,), BATCH, jnp.int32).at[d].set(
            jnp.arange(BATCH, dtype=jnp.int32))
    return jax.vmap(row)(dest)


def kernel(sparse_inputs, dense_inputs, fm_tables, lin_tables, W_dense,
           b_dense, bias, W1, b1, g1, bt1, W2, b2, g2, bt2, Wout, bout):
    i32 = jnp.int32
    vT = sparse_inputs.astype(i32).T
    d_t = dense_inputs.T
    tab3 = fm_tables.transpose(0, 2, 1)

    iota_b = jnp.broadcast_to(jnp.arange(BATCH, dtype=i32)[None, :],
                              (NUM_FIELDS, BATCH))
    vs, bs = lax.sort_key_val(vT, iota_b, dimension=1)
    c_of = jnp.minimum(vs // _CHUNK, _NCHUNK - 1)
    starts = jnp.cumsum(
        jnp.sum(jax.nn.one_hot(c_of, _NCHUNK, dtype=i32), axis=1),
        axis=1) - jnp.sum(jax.nn.one_hot(c_of, _NCHUNK, dtype=i32), axis=1)
    slot = (jnp.arange(BATCH, dtype=i32)[None, :]
            - jnp.take_along_axis(starts, c_of, axis=1))
    dest = c_of * _SLOTS + slot
    fbins = _NCHUNK * _SLOTS
    vsp = jnp.take_along_axis(
        jnp.concatenate([vs, jnp.full((NUM_FIELDS, 1), -1, i32)], axis=1),
        _inverse_perm(dest, fbins), axis=1)
    bsp = jnp.take_along_axis(
        jnp.concatenate([bs, jnp.zeros((NUM_FIELDS, 1), i32)], axis=1),
        _inverse_perm(dest, fbins), axis=1)

    x_t, lin_t = _sc_gather(tab3, lin_tables, vsp.reshape(-1),
                            bsp.reshape(-1))

    sel = jnp.tile(jnp.eye(EMB_DIM, dtype=jnp.float32), (1, NUM_FIELDS))
    cbias = (bias + b_dense + bout).reshape(1, 1)
    out = _tc_dense(
        x_t, d_t, lin_t, sel, W_dense, W1,
        b1.reshape(H1, 1), g1.reshape(H1, 1), bt1.reshape(H1, 1),
        W2, b2.reshape(H2, 1), g2.reshape(H2, 1), bt2.reshape(H2, 1),
        Wout, cbias)
    return out.reshape(BATCH)


# speedup vs baseline: 10.9892x; 10.9892x over previous
import functools

import jax
import jax.numpy as jnp
from jax import lax
from jax.experimental import pallas as pl
from jax.experimental.pallas import tpu as pltpu
from jax.experimental.pallas import tpu_sc as plsc

NUM_FIELDS = 26
VOCAB = 100000
EMB_DIM = 16
DENSE_DIM = 13
BATCH = 4096
H1, H2 = 64, 32
BN_EPS = 1e-5

PLANES = NUM_FIELDS * EMB_DIM
_NC, _NS = 2, 16
_CHUNK = 2048
_NCHUNK = 49
_TAIL_OFF = 48 * _CHUNK


_MAIN = 99968
_TAIL0 = VOCAB - 128
_UNITS = PLANES + NUM_FIELDS
_U_PER_W = 14


def _sc_gather(tab2, lin_tab, tab_tail, lin_tail, ids):
    mesh = plsc.VectorSubcoreMesh(core_axis_name="c", subcore_axis_name="s")

    @functools.partial(
        pl.kernel,
        mesh=mesh,
        compiler_params=pltpu.CompilerParams(needs_layout_passes=False),
        out_type=(
            jax.ShapeDtypeStruct((PLANES, BATCH), jnp.float32),
            jax.ShapeDtypeStruct((NUM_FIELDS, BATCH), jnp.float32),
        ),
        scratch_types=[
            pltpu.VMEM((_MAIN,), jnp.float32),
            pltpu.VMEM((128,), jnp.float32),
            pltpu.VMEM((BATCH,), jnp.int32),
            pltpu.VMEM((BATCH,), jnp.float32),
            pltpu.SemaphoreType.DMA,
        ],
    )
    def k(tab, ltab, ttail, ltail, ids_h, x_out, lin_out,
          buf, tbuf, idx_v, val_v, sem):
        wid = lax.axis_index("s") * _NC + lax.axis_index("c")

        def unit_body(j, _):
            u = wid * _U_PER_W + j

            @pl.when(u < _UNITS)
            def _():
                is_fm = u < PLANES
                f = jnp.where(is_fm, u // EMB_DIM, u - PLANES)
                pltpu.sync_copy(ids_h.at[pl.ds(f * BATCH, BATCH)], idx_v)

                @pl.when(is_fm)
                def _():
                    cp = pltpu.async_copy(
                        tab.at[u, pl.ds(0, _MAIN)], buf, sem)
                    pltpu.async_copy(ttail.at[u], tbuf, sem).wait()
                    cp.wait()

                @pl.when(jnp.logical_not(is_fm))
                def _():
                    lf = u - PLANES
                    cp = pltpu.async_copy(
                        ltab.at[lf, pl.ds(0, _MAIN)], buf, sem)
                    pltpu.async_copy(ltail.at[lf], tbuf, sem).wait()
                    cp.wait()

                for g in range(BATCH // 16):
                    v = idx_v[pl.ds(g * 16, 16)]
                    vmain = jnp.minimum(v, _MAIN - 1)
                    vtail = jnp.maximum(v - _TAIL0, 0)
                    vals = jnp.where(
                        v < _MAIN,
                        plsc.load_gather(buf, [vmain]),
                        plsc.load_gather(tbuf, [vtail]))
                    val_v[pl.ds(g * 16, 16)] = vals

                @pl.when(is_fm)
                def _():
                    pltpu.sync_copy(val_v, x_out.at[u])

                @pl.when(jnp.logical_not(is_fm))
                def _():
                    pltpu.sync_copy(val_v, lin_out.at[u - PLANES])

            return _

        lax.fori_loop(0, _U_PER_W, unit_body, None)

    return k(tab2, lin_tab, tab_tail, lin_tail, ids)


def _tc_body(x_ref, d_ref, lin_ref, sel_ref, wd_ref, w1_ref, b1_ref,
             g1_ref, bt1_ref, w2_ref, b2_ref, g2_ref, bt2_ref, wout_ref,
             cbias_ref, out_ref):
    f32 = jnp.float32
    x = x_ref[...]
    d = d_ref[...]
    sel = sel_ref[...]

    sv = jnp.dot(sel, x, preferred_element_type=f32)
    sq = jnp.dot(sel, x * x, preferred_element_type=f32)
    fm_logit = 0.5 * jnp.sum(sv * sv - sq, axis=0, keepdims=True)

    lin_logit = jnp.sum(lin_ref[...], axis=0, keepdims=True)
    lin_logit = lin_logit + jnp.dot(wd_ref[...], d,
                                    preferred_element_type=f32)

    inv = lax.rsqrt(jnp.float32(1.0 + BN_EPS))
    w1 = w1_ref[...]
    z = jnp.dot(w1[:, :PLANES], x, preferred_element_type=f32)
    z = z + jnp.dot(w1[:, PLANES:], d, preferred_element_type=f32)
    h = jnp.maximum((z + b1_ref[...]) * (g1_ref[...] * inv) + bt1_ref[...],
                    0.0)
    z2 = jnp.dot(w2_ref[...], h, preferred_element_type=f32)
    h2 = jnp.maximum((z2 + b2_ref[...]) * (g2_ref[...] * inv) + bt2_ref[...],
                     0.0)
    dnn_logit = jnp.dot(wout_ref[...], h2, preferred_element_type=f32)

    total = lin_logit + fm_logit + dnn_logit + cbias_ref[...]
    out_ref[...] = jax.nn.sigmoid(total)


def _tc_dense(x_t, d_t, lin_t, sel, wd, w1, b1, g1, bt1, w2, b2, g2, bt2,
              wout, cbias):
    bm = 1024
    grid = (BATCH // bm,)
    full = lambda shape: pl.BlockSpec(shape, lambda i: (0,) * len(shape))
    col = lambda rows: pl.BlockSpec((rows, bm), lambda i: (0, i))
    return pl.pallas_call(
        _tc_body,
        grid=grid,
        in_specs=[
            col(PLANES),
            col(DENSE_DIM),
            col(NUM_FIELDS),
            full((EMB_DIM, PLANES)),
            full((1, DENSE_DIM)),
            full((H1, PLANES + DENSE_DIM)),
            full((H1, 1)), full((H1, 1)), full((H1, 1)),
            full((H2, H1)),
            full((H2, 1)), full((H2, 1)), full((H2, 1)),
            full((1, H2)),
            full((1, 1)),
        ],
        out_specs=col(1),
        out_shape=jax.ShapeDtypeStruct((1, BATCH), jnp.float32),
    )(x_t, d_t, lin_t, sel, wd, w1, b1, g1, bt1, w2, b2, g2, bt2, wout,
      cbias)


def kernel(sparse_inputs, dense_inputs, fm_tables, lin_tables, W_dense,
           b_dense, bias, W1, b1, g1, bt1, W2, b2, g2, bt2, Wout, bout):
    i32 = jnp.int32
    vT = sparse_inputs.astype(i32).T
    d_t = dense_inputs.T
    tab2 = fm_tables.transpose(0, 2, 1).reshape(PLANES, VOCAB)
    tab_tail = tab2[:, _TAIL0:]
    lin_tail = lin_tables[:, _TAIL0:]

    x_t, lin_t = _sc_gather(tab2, lin_tables, tab_tail, lin_tail,
                            vT.reshape(-1))

    sel = jnp.tile(jnp.eye(EMB_DIM, dtype=jnp.float32), (1, NUM_FIELDS))
    cbias = (bias + b_dense + bout).reshape(1, 1)
    out = _tc_dense(
        x_t, d_t, lin_t, sel, W_dense, W1,
        b1.reshape(H1, 1), g1.reshape(H1, 1), bt1.reshape(H1, 1),
        W2, b2.reshape(H2, 1), g2.reshape(H2, 1), bt2.reshape(H2, 1),
        Wout, cbias)
    return out.reshape(BATCH)
